# baseline (speedup 1.0000x reference)
import functools

import jax
import jax.numpy as jnp
from jax import lax
from jax.experimental import pallas as pl
from jax.experimental.pallas import tpu as pltpu
from jax.experimental.pallas import tpu_sc as plsc

DIM = 16
BATCH = 16384
NUM_CORES = 2
NUM_SUBCORES = 16
NW = NUM_CORES * NUM_SUBCORES
CHUNK = 128
ROWS_PER_W = BATCH // (NW * CHUNK)
IDX_ROWS = BATCH // CHUNK


def _sc_gather_body(user_hbm, item_hbm, uemb_hbm, iemb_hbm, u_out, i_out,
                    uidx_v, iidx_v, urows_v, irows_v, sem):
    wid = lax.axis_index("s") * NUM_CORES + lax.axis_index("c")
    base = wid * ROWS_PER_W
    pltpu.sync_copy(user_hbm.at[pl.ds(base, ROWS_PER_W)], uidx_v)
    pltpu.sync_copy(item_hbm.at[pl.ds(base, ROWS_PER_W)], iidx_v)
    copies = []
    for j in range(ROWS_PER_W):
        copies.append(pltpu.async_copy(uemb_hbm.at[uidx_v.at[j]], urows_v.at[j], sem))
        copies.append(pltpu.async_copy(iemb_hbm.at[iidx_v.at[j]], irows_v.at[j], sem))
    for c in copies:
        c.wait()
    pltpu.sync_copy(urows_v, u_out.at[pl.ds(base, ROWS_PER_W)])
    pltpu.sync_copy(irows_v, i_out.at[pl.ds(base, ROWS_PER_W)])


_sc_gather = functools.partial(
    pl.kernel,
    out_type=(
        jax.ShapeDtypeStruct((IDX_ROWS, CHUNK, DIM), jnp.float32),
        jax.ShapeDtypeStruct((IDX_ROWS, CHUNK, DIM), jnp.float32),
    ),
    mesh=plsc.VectorSubcoreMesh(core_axis_name="c", subcore_axis_name="s"),
    compiler_params=pltpu.CompilerParams(use_tc_tiling_on_sc=False),
    scratch_types=[
        pltpu.VMEM((ROWS_PER_W, CHUNK), jnp.int32),
        pltpu.VMEM((ROWS_PER_W, CHUNK), jnp.int32),
        pltpu.VMEM((ROWS_PER_W, CHUNK, DIM), jnp.float32),
        pltpu.VMEM((ROWS_PER_W, CHUNK, DIM), jnp.float32),
        pltpu.SemaphoreType.DMA,
    ],
)(_sc_gather_body)


def _mlp_body(u_ref, i_ref, w1u_ref, w1i_ref, b1_ref, w2_ref, b2_ref, out_ref):
    h = jnp.dot(u_ref[...], w1u_ref[...], preferred_element_type=jnp.float32)
    h = h + jnp.dot(i_ref[...], w1i_ref[...], preferred_element_type=jnp.float32)
    h = jnp.maximum(h + b1_ref[...], 0.0)
    out_ref[...] = jnp.dot(h, w2_ref[...], preferred_element_type=jnp.float32) + b2_ref[0, 0]


def _mlp(u, i, w1u, w1i, b1, w2, b2):
    return pl.pallas_call(
        _mlp_body,
        out_shape=jax.ShapeDtypeStruct((BATCH, 1), jnp.float32),
    )(u, i, w1u, w1i, b1, w2, b2)


def kernel(user, item, user_emb, item_emb, W1, b1, W2, b2):
    user = user.astype(jnp.int32).reshape(IDX_ROWS, CHUNK)
    item = item.astype(jnp.int32).reshape(IDX_ROWS, CHUNK)
    u, i = _sc_gather(user, item, user_emb, item_emb)
    u = u.reshape(BATCH, DIM)
    i = i.reshape(BATCH, DIM)
    w1u = W1[:, :DIM].T
    w1i = W1[:, DIM:].T
    out = _mlp(u, i, w1u, w1i, b1.reshape(1, 32), W2.T, b2.reshape(1, 1))
    return out.reshape(BATCH)
